# Initial kernel scaffold; baseline (speedup 1.0000x reference)
import jax
import jax.numpy as jnp
from jax import lax
from jax.experimental import pallas as pl
from jax.experimental.pallas import tpu as pltpu
from jax.experimental.pallas import tpu_sc as plsc

B = 16384
D = 32
N_VALUES = B * 50
NW = 32
BAGS_PER_W = B // NW
CHUNK = 1024
NSTREAM = CHUNK // 128
OFFS_SLICE = 520
VPAD = N_VALUES + 2048


def _acc_rows(rows_ref, lo, hi, acc0, acc1):
    def body(j, carry):
        a0, a1 = carry
        a0 = a0 + rows_ref[j, pl.ds(0, 16)]
        a1 = a1 + rows_ref[j, pl.ds(16, 16)]
        return (a0, a1)
    return lax.fori_loop(lo, hi, body, (acc0, acc1))


def _gather_chunk(values2d, table, idx_ref, rows_ref, sem, c0):
    r0 = c0 // 128
    pltpu.sync_copy(values2d.at[pl.ds(r0, NSTREAM)], idx_ref)
    descs = []
    for j in range(NSTREAM):
        descs.append(pltpu.async_copy(
            table.at[idx_ref.at[j]],
            rows_ref.at[pl.ds(j * 128, 128)], sem))
    for d in descs:
        d.wait()


def _sc_body(values2d, offs_hbm, table, out_hbm, out2_hbm,
             idx_ref, rows_ref, out_loc, offs_s, tail_s, sem):
    wid = lax.axis_index("s") * 2 + lax.axis_index("c")
    bag_lo = wid * BAGS_PER_W
    bag_hi = jnp.minimum(bag_lo + BAGS_PER_W, B - 1)

    pltpu.sync_copy(offs_hbm.at[pl.ds(bag_lo, OFFS_SLICE)], offs_s)
    pltpu.sync_copy(offs_hbm.at[pl.ds(B - 8, 8)], tail_s)

    zero = jnp.zeros((16,), jnp.float32)

    s_w = offs_s[0]
    e_w = offs_s[bag_hi - bag_lo]
    a0 = (s_w // 128) * 128
    n_chunks = jnp.maximum((e_w - a0 + CHUNK - 1) // CHUNK, 1)

    def chunk_body(c, carry):
        b, acc0, acc1 = carry
        c0 = a0 + c * CHUNK
        c_end = c0 + CHUNK
        _gather_chunk(values2d, table, idx_ref, rows_ref, sem, c0)

        def w_cond(st):
            b_, _, _ = st
            return jnp.logical_and(b_ < bag_hi,
                                   offs_s[b_ + 1 - bag_lo] <= c_end)

        def w_body(st):
            b_, a0_, a1_ = st
            lo = jnp.maximum(offs_s[b_ - bag_lo], c0) - c0
            hi = offs_s[b_ + 1 - bag_lo] - c0
            a0_, a1_ = _acc_rows(rows_ref, lo, hi, a0_, a1_)
            out_loc[b_ - bag_lo, pl.ds(0, 16)] = a0_
            out_loc[b_ - bag_lo, pl.ds(16, 16)] = a1_
            return (b_ + 1, zero, zero)

        b, acc0, acc1 = lax.while_loop(w_cond, w_body, (b, acc0, acc1))

        lo = jnp.maximum(offs_s[b - bag_lo], c0) - c0
        acc0, acc1 = _acc_rows(rows_ref, lo, CHUNK, acc0, acc1)
        return (b, acc0, acc1)

    lax.fori_loop(0, n_chunks, chunk_body, (bag_lo, zero, zero))
    pltpu.sync_copy(out_loc, out_hbm.at[pl.ds(bag_lo, BAGS_PER_W)])

    t0 = tail_s[7]
    a0t = (t0 // 128) * 128
    q = ((N_VALUES - a0t + NW * 128 - 1) // (NW * 128)) * 128
    lo_w = jnp.maximum(a0t + wid * q, t0)
    hi_w = jnp.minimum(a0t + (wid + 1) * q, N_VALUES)
    base = (lo_w // 128) * 128
    n_ch2 = jnp.maximum((hi_w - base + CHUNK - 1) // CHUNK, 0)

    def chunk2_body(c, carry):
        acc0, acc1 = carry
        c0 = base + c * CHUNK
        _gather_chunk(values2d, table, idx_ref, rows_ref, sem, c0)
        lo = jnp.maximum(lo_w, c0) - c0
        hi = jnp.minimum(hi_w, c0 + CHUNK) - c0
        return _acc_rows(rows_ref, lo, hi, acc0, acc1)

    acc0, acc1 = lax.fori_loop(0, n_ch2, chunk2_body, (zero, zero))
    out_loc[0, pl.ds(0, 16)] = acc0
    out_loc[0, pl.ds(16, 16)] = acc1
    pltpu.sync_copy(out_loc.at[pl.ds(0, 1)], out2_hbm.at[pl.ds(wid, 1)])


@jax.jit
def kernel(values, lengths, table):
    offsets = jnp.concatenate(
        [jnp.zeros((1,), jnp.int32), jnp.cumsum(lengths, dtype=jnp.int32)])
    offs_pad = jnp.pad(offsets, (0, (B - 8 + OFFS_SLICE) - (B + 1)),
                       mode="edge")
    values2d = jnp.pad(values, (0, VPAD - N_VALUES)).reshape(VPAD // 128, 128)

    mesh = plsc.VectorSubcoreMesh(core_axis_name="c", subcore_axis_name="s")
    f = pl.kernel(
        _sc_body,
        out_type=[jax.ShapeDtypeStruct((B, D), jnp.float32),
                  jax.ShapeDtypeStruct((NW, D), jnp.float32)],
        mesh=mesh,
        scratch_types=[
            pltpu.VMEM((NSTREAM, 128), jnp.int32),
            pltpu.VMEM((CHUNK, D), jnp.float32),
            pltpu.VMEM((BAGS_PER_W, D), jnp.float32),
            pltpu.SMEM((OFFS_SLICE,), jnp.int32),
            pltpu.SMEM((8,), jnp.int32),
            pltpu.SemaphoreType.DMA,
        ],
    )
    out_main, out2 = f(values2d, offs_pad, table)
    return out_main.at[B - 1].set(jnp.sum(out2, axis=0))


# speedup vs baseline: 164.1281x; 164.1281x over previous
import jax
import jax.numpy as jnp
from jax import lax
from jax.experimental import pallas as pl
from jax.experimental.pallas import tpu as pltpu
from jax.experimental.pallas import tpu_sc as plsc

B = 16384
D = 32
N_VALUES = B * 50
NW = 32
BAGS_PER_W = B // NW
CHUNK = 1024
NSTREAM = CHUNK // 128
OFFS_SLICE = 536
VPAD = N_VALUES + 2048


def _sread(ref, i):
    return ref[pl.ds(i, 16)][0]


def _acc_rows(rows_ref, lo, hi, acc0, acc1):
    def body(j, carry):
        a0, a1 = carry
        a0 = a0 + rows_ref[j, pl.ds(0, 16)]
        a1 = a1 + rows_ref[j, pl.ds(16, 16)]
        return (a0, a1)
    return lax.fori_loop(lo, hi, body, (acc0, acc1))


def _gather_chunk(values2d, table, idx_ref, rows_ref, sem, c0):
    r0 = pl.multiple_of(c0 // 128, NSTREAM)
    pltpu.sync_copy(values2d.at[pl.ds(r0, NSTREAM)], idx_ref)
    descs = []
    for j in range(NSTREAM):
        descs.append(pltpu.async_copy(
            table.at[idx_ref.at[j]],
            rows_ref.at[pl.ds(j * 128, 128)], sem))
    for d in descs:
        d.wait()


def _sc_body(values2d, offs_hbm, table, out_hbm, out2_hbm,
             idx_ref, rows_ref, out_loc, offs_v, tail_v, sem):
    wid = lax.axis_index("s") * 2 + lax.axis_index("c")
    bag_lo = wid * BAGS_PER_W
    bag_hi = jnp.minimum(bag_lo + BAGS_PER_W, B - 1)

    pltpu.sync_copy(offs_hbm.at[pl.ds(bag_lo, OFFS_SLICE)], offs_v)
    pltpu.sync_copy(offs_hbm.at[pl.ds(B - 16, 16)], tail_v)

    zero = jnp.zeros((16,), jnp.float32)

    s_w = _sread(offs_v, 0)
    e_w = _sread(offs_v, bag_hi - bag_lo)
    a0 = (s_w // CHUNK) * CHUNK
    n_chunks = jnp.maximum((e_w - a0 + CHUNK - 1) // CHUNK, 1)

    n_loc = bag_hi - bag_lo

    def chunk_body(c, carry):
        b, acc0, acc1 = carry
        c0 = a0 + c * CHUNK
        c_end = c0 + CHUNK
        _gather_chunk(values2d, table, idx_ref, rows_ref, sem, c0)

        def bs_body(_, st):
            lo_, hi_ = st
            mid = (lo_ + hi_) // 2
            le = _sread(offs_v, mid) <= c_end
            go = lo_ < hi_
            lo_ = jnp.where(jnp.logical_and(go, le), mid + 1, lo_)
            hi_ = jnp.where(jnp.logical_and(go, jnp.logical_not(le)),
                            mid, hi_)
            return (lo_, hi_)
        ub, _ = lax.fori_loop(0, 10, bs_body, (jnp.int32(0), jnp.int32(513)))

        b_stop = jnp.minimum(ub - 1, n_loc)

        def fin_body(ib, st):
            a0_, a1_ = st
            lo = jnp.maximum(_sread(offs_v, ib), c0) - c0
            hi = _sread(offs_v, ib + 1) - c0
            a0_, a1_ = _acc_rows(rows_ref, lo, hi, a0_, a1_)
            out_loc[ib, pl.ds(0, 16)] = a0_
            out_loc[ib, pl.ds(16, 16)] = a1_
            return (zero, zero)

        acc0, acc1 = lax.fori_loop(b, b_stop, fin_body, (acc0, acc1))
        b = jnp.maximum(b, b_stop)

        lo = jnp.maximum(_sread(offs_v, jnp.minimum(b, n_loc)), c0) - c0
        acc0, acc1 = _acc_rows(rows_ref, lo, CHUNK, acc0, acc1)
        return (b, acc0, acc1)

    lax.fori_loop(0, n_chunks, chunk_body, (jnp.int32(0), zero, zero))
    pltpu.sync_copy(out_loc, out_hbm.at[pl.ds(bag_lo, BAGS_PER_W)])

    t0 = tail_v[...][15]
    a0t = (t0 // 128) * 128
    q = ((N_VALUES - a0t + NW * 128 - 1) // (NW * 128)) * 128
    lo_w = jnp.maximum(a0t + wid * q, t0)
    hi_w = jnp.minimum(a0t + (wid + 1) * q, N_VALUES)
    base = (lo_w // CHUNK) * CHUNK
    n_ch2 = jnp.maximum((hi_w - base + CHUNK - 1) // CHUNK, 0)

    def chunk2_body(c, carry):
        acc0, acc1 = carry
        c0 = base + c * CHUNK
        _gather_chunk(values2d, table, idx_ref, rows_ref, sem, c0)
        lo = jnp.maximum(lo_w, c0) - c0
        hi = jnp.minimum(hi_w, c0 + CHUNK) - c0
        return _acc_rows(rows_ref, lo, hi, acc0, acc1)

    acc0, acc1 = lax.fori_loop(0, n_ch2, chunk2_body, (zero, zero))
    out_loc[0, pl.ds(0, 16)] = acc0
    out_loc[0, pl.ds(16, 16)] = acc1
    pltpu.sync_copy(out_loc.at[pl.ds(0, 1)], out2_hbm.at[pl.ds(wid, 1)])


@jax.jit
def kernel(values, lengths, table):
    offsets = jnp.concatenate(
        [jnp.zeros((1,), jnp.int32), jnp.cumsum(lengths, dtype=jnp.int32)])
    offs_pad = jnp.pad(offsets, (0, (B - 8 + OFFS_SLICE) - (B + 1)),
                       mode="edge")
    values2d = jnp.pad(values, (0, VPAD - N_VALUES)).reshape(VPAD // 128, 128)

    mesh = plsc.VectorSubcoreMesh(core_axis_name="c", subcore_axis_name="s")
    f = pl.kernel(
        _sc_body,
        out_type=[jax.ShapeDtypeStruct((B, D), jnp.float32),
                  jax.ShapeDtypeStruct((NW, D), jnp.float32)],
        mesh=mesh,
        compiler_params=pltpu.CompilerParams(use_tc_tiling_on_sc=False),
        scratch_types=[
            pltpu.VMEM((NSTREAM, 128), jnp.int32),
            pltpu.VMEM((CHUNK, D), jnp.float32),
            pltpu.VMEM((BAGS_PER_W, D), jnp.float32),
            pltpu.VMEM((OFFS_SLICE,), jnp.int32),
            pltpu.VMEM((16,), jnp.int32),
            pltpu.SemaphoreType.DMA,
        ],
    )
    out_main, out2 = f(values2d, offs_pad, table)
    return out_main.at[B - 1].set(jnp.sum(out2, axis=0))


# speedup vs baseline: 173.3996x; 1.0565x over previous
import jax
import jax.numpy as jnp
from jax import lax
from jax.experimental import pallas as pl
from jax.experimental.pallas import tpu as pltpu
from jax.experimental.pallas import tpu_sc as plsc

B = 16384
D = 32
N_VALUES = B * 50
NW = 32
BAGS_PER_W = B // NW
CHUNK = 1024
NSTREAM = CHUNK // 128
OFFS_SLICE = 536
VPAD = N_VALUES + 4096


def _sread(ref, i):
    return ref[pl.ds(i, 16)][0]


def _acc_rows(rows_ref, lo, hi, acc0, acc1):
    def body(j, carry):
        a0, a1 = carry
        a0 = a0 + rows_ref[j, pl.ds(0, 16)]
        a1 = a1 + rows_ref[j, pl.ds(16, 16)]
        return (a0, a1)
    return lax.fori_loop(lo, hi, body, (acc0, acc1))


def _issue_chunk(values2d, table, idx_ref, rows_ref, sem, c0):
    r0 = pl.multiple_of(c0 // 128, NSTREAM)
    pltpu.sync_copy(values2d.at[pl.ds(r0, NSTREAM)], idx_ref)
    for j in range(NSTREAM):
        pltpu.async_copy(table.at[idx_ref.at[j]],
                         rows_ref.at[pl.ds(j * 128, 128)], sem)


def _wait_chunk(table, idx_ref, rows_ref, sem):
    for j in range(NSTREAM):
        pltpu.make_async_copy(table.at[idx_ref.at[j]],
                              rows_ref.at[pl.ds(j * 128, 128)], sem).wait()


def _sc_body(values2d, offs_hbm, table, out_hbm, out2_hbm,
             idx_a, idx_b, rows_a, rows_b, out_loc, offs_v, tail_v,
             sem_a, sem_b):
    wid = lax.axis_index("s") * 2 + lax.axis_index("c")
    bag_lo = wid * BAGS_PER_W
    bag_hi = jnp.minimum(bag_lo + BAGS_PER_W, B - 1)

    pltpu.sync_copy(offs_hbm.at[pl.ds(bag_lo, OFFS_SLICE)], offs_v)
    pltpu.sync_copy(offs_hbm.at[pl.ds(B - 16, 16)], tail_v)

    zero = jnp.zeros((16,), jnp.float32)

    s_w = _sread(offs_v, 0)
    e_w = _sread(offs_v, bag_hi - bag_lo)
    a0 = (s_w // CHUNK) * CHUNK
    n_chunks = jnp.maximum((e_w - a0 + CHUNK - 1) // CHUNK, 1)
    n_loc = bag_hi - bag_lo

    def process1(c, rows_ref, carry):
        b, acc0, acc1 = carry
        c0 = a0 + c * CHUNK
        c_end = c0 + CHUNK

        def bs_body(_, st):
            lo_, hi_ = st
            mid = (lo_ + hi_) // 2
            le = _sread(offs_v, mid) <= c_end
            go = lo_ < hi_
            lo_ = jnp.where(jnp.logical_and(go, le), mid + 1, lo_)
            hi_ = jnp.where(jnp.logical_and(go, jnp.logical_not(le)),
                            mid, hi_)
            return (lo_, hi_)
        ub, _ = lax.fori_loop(0, 10, bs_body, (jnp.int32(0), jnp.int32(513)))

        b_stop = jnp.minimum(ub - 1, n_loc)

        def fin_body(ib, st):
            a0_, a1_ = st
            lo = jnp.maximum(_sread(offs_v, ib), c0) - c0
            hi = _sread(offs_v, ib + 1) - c0
            a0_, a1_ = _acc_rows(rows_ref, lo, hi, a0_, a1_)
            out_loc[ib, pl.ds(0, 16)] = a0_
            out_loc[ib, pl.ds(16, 16)] = a1_
            return (zero, zero)

        acc0, acc1 = lax.fori_loop(b, b_stop, fin_body, (acc0, acc1))
        b = jnp.maximum(b, b_stop)

        lo = jnp.maximum(_sread(offs_v, jnp.minimum(b, n_loc)), c0) - c0
        acc0, acc1 = _acc_rows(rows_ref, lo, CHUNK, acc0, acc1)
        return (b, acc0, acc1)

    def run_pipeline(n_real, c0_of, process, carry):
        _issue_chunk(values2d, table, idx_a, rows_a, sem_a, c0_of(0))

        def pair_body(ci, carry):
            c = 2 * ci
            _wait_chunk(table, idx_a, rows_a, sem_a)
            _issue_chunk(values2d, table, idx_b, rows_b, sem_b, c0_of(c + 1))
            carry = process(c, rows_a, carry)
            _wait_chunk(table, idx_b, rows_b, sem_b)
            _issue_chunk(values2d, table, idx_a, rows_a, sem_a, c0_of(c + 2))
            carry = process(c + 1, rows_b, carry)
            return carry

        n_pairs = (n_real + 1) // 2
        carry = lax.fori_loop(0, n_pairs, pair_body, carry)
        _wait_chunk(table, idx_a, rows_a, sem_a)
        return carry

    run_pipeline(n_chunks, lambda c: a0 + c * CHUNK, process1,
                 (jnp.int32(0), zero, zero))
    pltpu.sync_copy(out_loc, out_hbm.at[pl.ds(bag_lo, BAGS_PER_W)])

    t0 = tail_v[...][15]
    a0t = (t0 // 128) * 128
    q = ((N_VALUES - a0t + NW * 128 - 1) // (NW * 128)) * 128
    lo_w = jnp.minimum(jnp.maximum(a0t + wid * q, t0), N_VALUES)
    hi_w = jnp.minimum(a0t + (wid + 1) * q, N_VALUES)
    base = (lo_w // CHUNK) * CHUNK
    n_ch2 = jnp.maximum((hi_w - base + CHUNK - 1) // CHUNK, 0)

    def process2(c, rows_ref, carry):
        acc0, acc1 = carry
        c0 = base + c * CHUNK
        lo = jnp.maximum(lo_w, c0) - c0
        hi = jnp.minimum(jnp.maximum(hi_w - c0, lo), CHUNK)
        return _acc_rows(rows_ref, lo, hi, acc0, acc1)

    acc0, acc1 = run_pipeline(n_ch2, lambda c: base + c * CHUNK, process2,
                              (zero, zero))
    out_loc[0, pl.ds(0, 16)] = acc0
    out_loc[0, pl.ds(16, 16)] = acc1
    pltpu.sync_copy(out_loc.at[pl.ds(0, 1)], out2_hbm.at[pl.ds(wid, 1)])


@jax.jit
def kernel(values, lengths, table):
    offsets = jnp.concatenate(
        [jnp.zeros((1,), jnp.int32), jnp.cumsum(lengths, dtype=jnp.int32)])
    offs_pad = jnp.pad(offsets, (0, (B - 8 + OFFS_SLICE) - (B + 1)),
                       mode="edge")
    values2d = jnp.pad(values, (0, VPAD - N_VALUES)).reshape(VPAD // 128, 128)

    mesh = plsc.VectorSubcoreMesh(core_axis_name="c", subcore_axis_name="s")
    f = pl.kernel(
        _sc_body,
        out_type=[jax.ShapeDtypeStruct((B, D), jnp.float32),
                  jax.ShapeDtypeStruct((NW, D), jnp.float32)],
        mesh=mesh,
        compiler_params=pltpu.CompilerParams(use_tc_tiling_on_sc=False),
        scratch_types=[
            pltpu.VMEM((NSTREAM, 128), jnp.int32),
            pltpu.VMEM((NSTREAM, 128), jnp.int32),
            pltpu.VMEM((CHUNK, D), jnp.float32),
            pltpu.VMEM((CHUNK, D), jnp.float32),
            pltpu.VMEM((BAGS_PER_W, D), jnp.float32),
            pltpu.VMEM((OFFS_SLICE,), jnp.int32),
            pltpu.VMEM((16,), jnp.int32),
            pltpu.SemaphoreType.DMA,
            pltpu.SemaphoreType.DMA,
        ],
    )
    out_main, out2 = f(values2d, offs_pad, table)
    return out_main.at[B - 1].set(jnp.sum(out2, axis=0))


# speedup vs baseline: 173.7494x; 1.0020x over previous
import jax
import jax.numpy as jnp
from jax import lax
from jax.experimental import pallas as pl
from jax.experimental.pallas import tpu as pltpu
from jax.experimental.pallas import tpu_sc as plsc

B = 16384
D = 32
N_VALUES = B * 50
NW = 32
BAGS_PER_W = B // NW
CHUNK = 1024
NSTREAM = CHUNK // 128
OFFS_SLICE = 536
VROWS = N_VALUES // 128


def _sread(ref, i):
    return ref[pl.ds(i, 16)][0]


def _acc_rows(rows_ref, lo, hi, acc0, acc1):
    def body(j, carry):
        a0, a1 = carry
        a0 = a0 + rows_ref[j, pl.ds(0, 16)]
        a1 = a1 + rows_ref[j, pl.ds(16, 16)]
        return (a0, a1)
    return lax.fori_loop(lo, hi, body, (acc0, acc1))


def _issue_chunk(values2d, table, idx_ref, rows_ref, sem, c0):
    r0 = pl.multiple_of(jnp.minimum(c0 // 128, VROWS - NSTREAM), NSTREAM)
    pltpu.sync_copy(values2d.at[pl.ds(r0, NSTREAM)], idx_ref)
    for j in range(NSTREAM):
        pltpu.async_copy(table.at[idx_ref.at[j]],
                         rows_ref.at[pl.ds(j * 128, 128)], sem)


def _wait_chunk(table, idx_ref, rows_ref, sem):
    for j in range(NSTREAM):
        pltpu.make_async_copy(table.at[idx_ref.at[j]],
                              rows_ref.at[pl.ds(j * 128, 128)], sem).wait()


def _sc_body(values2d, offs_hbm, table, out_hbm, out2_hbm,
             idx_a, idx_b, rows_a, rows_b, out_loc, offs_v, tail_v,
             sem_a, sem_b):
    wid = lax.axis_index("s") * 2 + lax.axis_index("c")
    bag_lo = wid * BAGS_PER_W
    bag_hi = jnp.minimum(bag_lo + BAGS_PER_W, B - 1)

    pltpu.sync_copy(offs_hbm.at[pl.ds(bag_lo, OFFS_SLICE)], offs_v)
    pltpu.sync_copy(offs_hbm.at[pl.ds(B - 16, 16)], tail_v)

    zero = jnp.zeros((16,), jnp.float32)

    s_w = _sread(offs_v, 0)
    e_w = _sread(offs_v, bag_hi - bag_lo)
    a0 = (s_w // CHUNK) * CHUNK
    n_chunks = jnp.maximum((e_w - a0 + CHUNK - 1) // CHUNK, 1)
    n_loc = bag_hi - bag_lo

    def process1(c, rows_ref, carry):
        b, acc0, acc1 = carry
        c0 = a0 + c * CHUNK
        c_end = c0 + CHUNK

        def bs_body(_, st):
            lo_, hi_ = st
            mid = (lo_ + hi_) // 2
            le = _sread(offs_v, mid) <= c_end
            go = lo_ < hi_
            lo_ = jnp.where(jnp.logical_and(go, le), mid + 1, lo_)
            hi_ = jnp.where(jnp.logical_and(go, jnp.logical_not(le)),
                            mid, hi_)
            return (lo_, hi_)
        ub, _ = lax.fori_loop(0, 10, bs_body, (jnp.int32(0), jnp.int32(513)))

        b_stop = jnp.minimum(ub - 1, n_loc)

        def fin_body(ib, st):
            a0_, a1_ = st
            lo = jnp.maximum(_sread(offs_v, ib), c0) - c0
            hi = _sread(offs_v, ib + 1) - c0
            a0_, a1_ = _acc_rows(rows_ref, lo, hi, a0_, a1_)
            out_loc[ib, pl.ds(0, 16)] = a0_
            out_loc[ib, pl.ds(16, 16)] = a1_
            return (zero, zero)

        acc0, acc1 = lax.fori_loop(b, b_stop, fin_body, (acc0, acc1))
        b = jnp.maximum(b, b_stop)

        lo = jnp.maximum(_sread(offs_v, jnp.minimum(b, n_loc)), c0) - c0
        acc0, acc1 = _acc_rows(rows_ref, lo, CHUNK, acc0, acc1)
        return (b, acc0, acc1)

    def run_pipeline(n_real, c0_of, process, carry):
        _issue_chunk(values2d, table, idx_a, rows_a, sem_a, c0_of(0))

        def pair_body(ci, carry):
            c = 2 * ci
            _wait_chunk(table, idx_a, rows_a, sem_a)
            _issue_chunk(values2d, table, idx_b, rows_b, sem_b, c0_of(c + 1))
            carry = process(c, rows_a, carry)
            _wait_chunk(table, idx_b, rows_b, sem_b)
            _issue_chunk(values2d, table, idx_a, rows_a, sem_a, c0_of(c + 2))
            carry = process(c + 1, rows_b, carry)
            return carry

        n_pairs = (n_real + 1) // 2
        carry = lax.fori_loop(0, n_pairs, pair_body, carry)
        _wait_chunk(table, idx_a, rows_a, sem_a)
        return carry

    run_pipeline(n_chunks, lambda c: a0 + c * CHUNK, process1,
                 (jnp.int32(0), zero, zero))
    pltpu.sync_copy(out_loc, out_hbm.at[pl.ds(bag_lo, BAGS_PER_W)])

    t0 = tail_v[...][15]
    a0t = (t0 // 128) * 128
    q = ((N_VALUES - a0t + NW * 128 - 1) // (NW * 128)) * 128
    lo_w = jnp.minimum(jnp.maximum(a0t + wid * q, t0), N_VALUES)
    hi_w = jnp.minimum(a0t + (wid + 1) * q, N_VALUES)
    base = (lo_w // CHUNK) * CHUNK
    n_ch2 = jnp.maximum((hi_w - base + CHUNK - 1) // CHUNK, 0)

    def process2(c, rows_ref, carry):
        acc0, acc1 = carry
        c0 = base + c * CHUNK
        lo = jnp.maximum(lo_w, c0) - c0
        hi = jnp.minimum(jnp.maximum(hi_w - c0, lo), CHUNK)
        return _acc_rows(rows_ref, lo, hi, acc0, acc1)

    acc0, acc1 = run_pipeline(n_ch2, lambda c: base + c * CHUNK, process2,
                              (zero, zero))
    out_loc[0, pl.ds(0, 16)] = acc0
    out_loc[0, pl.ds(16, 16)] = acc1
    pltpu.sync_copy(out_loc.at[pl.ds(0, 1)], out2_hbm.at[pl.ds(wid, 1)])


@jax.jit
def kernel(values, lengths, table):
    offsets = jnp.concatenate(
        [jnp.zeros((1,), jnp.int32), jnp.cumsum(lengths, dtype=jnp.int32)])
    offs_pad = jnp.pad(offsets, (0, (B - 8 + OFFS_SLICE) - (B + 1)),
                       mode="edge")
    values2d = values.reshape(VROWS, 128)

    mesh = plsc.VectorSubcoreMesh(core_axis_name="c", subcore_axis_name="s")
    f = pl.kernel(
        _sc_body,
        out_type=[jax.ShapeDtypeStruct((B, D), jnp.float32),
                  jax.ShapeDtypeStruct((NW, D), jnp.float32)],
        mesh=mesh,
        compiler_params=pltpu.CompilerParams(use_tc_tiling_on_sc=False),
        scratch_types=[
            pltpu.VMEM((NSTREAM, 128), jnp.int32),
            pltpu.VMEM((NSTREAM, 128), jnp.int32),
            pltpu.VMEM((CHUNK, D), jnp.float32),
            pltpu.VMEM((CHUNK, D), jnp.float32),
            pltpu.VMEM((BAGS_PER_W, D), jnp.float32),
            pltpu.VMEM((OFFS_SLICE,), jnp.int32),
            pltpu.VMEM((16,), jnp.int32),
            pltpu.SemaphoreType.DMA,
            pltpu.SemaphoreType.DMA,
        ],
    )
    out_main, out2 = f(values2d, offs_pad, table)
    return out_main.at[B - 1].set(jnp.sum(out2, axis=0))


# speedup vs baseline: 197.4289x; 1.1363x over previous
import jax
import jax.numpy as jnp
from jax import lax
from jax.experimental import pallas as pl
from jax.experimental.pallas import tpu as pltpu
from jax.experimental.pallas import tpu_sc as plsc

B = 16384
D = 32
N_VALUES = B * 50
NW = 32
BAGS_PER_W = B // NW
CHUNK = 1024
NSTREAM = CHUNK // 128
OFFS_SLICE = 536
VROWS = N_VALUES // 128


def _sread(ref, i):
    return ref[pl.ds(i, 16)][0]


def _acc_rows(rows_ref, lo, hi, acc0, acc1):
    n4 = (hi - lo) // 4

    def body4(g, carry):
        a0, a1 = carry
        j = lo + g * 4
        s0 = ((rows_ref[j, pl.ds(0, 16)] + rows_ref[j + 1, pl.ds(0, 16)])
              + (rows_ref[j + 2, pl.ds(0, 16)] + rows_ref[j + 3, pl.ds(0, 16)]))
        s1 = ((rows_ref[j, pl.ds(16, 16)] + rows_ref[j + 1, pl.ds(16, 16)])
              + (rows_ref[j + 2, pl.ds(16, 16)] + rows_ref[j + 3, pl.ds(16, 16)]))
        return (a0 + s0, a1 + s1)

    acc0, acc1 = lax.fori_loop(0, n4, body4, (acc0, acc1))

    def body1(j, carry):
        a0, a1 = carry
        a0 = a0 + rows_ref[j, pl.ds(0, 16)]
        a1 = a1 + rows_ref[j, pl.ds(16, 16)]
        return (a0, a1)
    return lax.fori_loop(lo + n4 * 4, hi, body1, (acc0, acc1))


def _issue_chunk(values2d, table, idx_ref, rows_ref, sem, c0):
    r0 = pl.multiple_of(jnp.minimum(c0 // 128, VROWS - NSTREAM), NSTREAM)
    pltpu.sync_copy(values2d.at[pl.ds(r0, NSTREAM)], idx_ref)
    for j in range(NSTREAM):
        pltpu.async_copy(table.at[idx_ref.at[j]],
                         rows_ref.at[pl.ds(j * 128, 128)], sem)


def _wait_chunk(table, idx_ref, rows_ref, sem):
    for j in range(NSTREAM):
        pltpu.make_async_copy(table.at[idx_ref.at[j]],
                              rows_ref.at[pl.ds(j * 128, 128)], sem).wait()


def _sc_body(values2d, offs_hbm, table, out_hbm, out2_hbm,
             idx_a, idx_b, rows_a, rows_b, out_loc, offs_v, tail_v,
             sem_a, sem_b):
    wid = lax.axis_index("s") * 2 + lax.axis_index("c")
    bag_lo = wid * BAGS_PER_W
    bag_hi = jnp.minimum(bag_lo + BAGS_PER_W, B - 1)

    pltpu.sync_copy(offs_hbm.at[pl.ds(bag_lo, OFFS_SLICE)], offs_v)
    pltpu.sync_copy(offs_hbm.at[pl.ds(B - 16, 16)], tail_v)

    zero = jnp.zeros((16,), jnp.float32)

    s_w = _sread(offs_v, 0)
    e_w = _sread(offs_v, bag_hi - bag_lo)
    a0 = (s_w // CHUNK) * CHUNK
    n_chunks = jnp.maximum((e_w - a0 + CHUNK - 1) // CHUNK, 1)
    n_loc = bag_hi - bag_lo

    def process1(c, rows_ref, carry):
        b, acc0, acc1 = carry
        c0 = a0 + c * CHUNK
        c_end = c0 + CHUNK

        def bs_body(_, st):
            lo_, hi_ = st
            mid = (lo_ + hi_) // 2
            le = _sread(offs_v, mid) <= c_end
            go = lo_ < hi_
            lo_ = jnp.where(jnp.logical_and(go, le), mid + 1, lo_)
            hi_ = jnp.where(jnp.logical_and(go, jnp.logical_not(le)),
                            mid, hi_)
            return (lo_, hi_)
        ub, _ = lax.fori_loop(0, 10, bs_body, (jnp.int32(0), jnp.int32(513)))

        b_stop = jnp.minimum(ub - 1, n_loc)

        def fin_body(ib, st):
            a0_, a1_ = st
            lo = jnp.maximum(_sread(offs_v, ib), c0) - c0
            hi = _sread(offs_v, ib + 1) - c0
            a0_, a1_ = _acc_rows(rows_ref, lo, hi, a0_, a1_)
            out_loc[ib, pl.ds(0, 16)] = a0_
            out_loc[ib, pl.ds(16, 16)] = a1_
            return (zero, zero)

        acc0, acc1 = lax.fori_loop(b, b_stop, fin_body, (acc0, acc1))
        b = jnp.maximum(b, b_stop)

        lo = jnp.maximum(_sread(offs_v, jnp.minimum(b, n_loc)), c0) - c0
        acc0, acc1 = _acc_rows(rows_ref, lo, CHUNK, acc0, acc1)
        return (b, acc0, acc1)

    def run_pipeline(n_real, c0_of, process, carry):
        _issue_chunk(values2d, table, idx_a, rows_a, sem_a, c0_of(0))

        def pair_body(ci, carry):
            c = 2 * ci
            _wait_chunk(table, idx_a, rows_a, sem_a)
            _issue_chunk(values2d, table, idx_b, rows_b, sem_b, c0_of(c + 1))
            carry = process(c, rows_a, carry)
            _wait_chunk(table, idx_b, rows_b, sem_b)
            _issue_chunk(values2d, table, idx_a, rows_a, sem_a, c0_of(c + 2))
            carry = process(c + 1, rows_b, carry)
            return carry

        n_pairs = (n_real + 1) // 2
        carry = lax.fori_loop(0, n_pairs, pair_body, carry)
        _wait_chunk(table, idx_a, rows_a, sem_a)
        return carry

    run_pipeline(n_chunks, lambda c: a0 + c * CHUNK, process1,
                 (jnp.int32(0), zero, zero))
    pltpu.sync_copy(out_loc, out_hbm.at[pl.ds(bag_lo, BAGS_PER_W)])

    t0 = tail_v[...][15]
    a0t = (t0 // 128) * 128
    q = ((N_VALUES - a0t + NW * 128 - 1) // (NW * 128)) * 128
    lo_w = jnp.minimum(jnp.maximum(a0t + wid * q, t0), N_VALUES)
    hi_w = jnp.minimum(a0t + (wid + 1) * q, N_VALUES)
    base = (lo_w // CHUNK) * CHUNK
    n_ch2 = jnp.maximum((hi_w - base + CHUNK - 1) // CHUNK, 0)

    def process2(c, rows_ref, carry):
        acc0, acc1 = carry
        c0 = base + c * CHUNK
        lo = jnp.maximum(lo_w, c0) - c0
        hi = jnp.minimum(jnp.maximum(hi_w - c0, lo), CHUNK)
        return _acc_rows(rows_ref, lo, hi, acc0, acc1)

    acc0, acc1 = run_pipeline(n_ch2, lambda c: base + c * CHUNK, process2,
                              (zero, zero))
    out_loc[0, pl.ds(0, 16)] = acc0
    out_loc[0, pl.ds(16, 16)] = acc1
    pltpu.sync_copy(out_loc.at[pl.ds(0, 1)], out2_hbm.at[pl.ds(wid, 1)])


@jax.jit
def kernel(values, lengths, table):
    offsets = jnp.concatenate(
        [jnp.zeros((1,), jnp.int32), jnp.cumsum(lengths, dtype=jnp.int32)])
    offs_pad = jnp.pad(offsets, (0, (B - 8 + OFFS_SLICE) - (B + 1)),
                       mode="edge")
    values2d = values.reshape(VROWS, 128)

    mesh = plsc.VectorSubcoreMesh(core_axis_name="c", subcore_axis_name="s")
    f = pl.kernel(
        _sc_body,
        out_type=[jax.ShapeDtypeStruct((B, D), jnp.float32),
                  jax.ShapeDtypeStruct((NW, D), jnp.float32)],
        mesh=mesh,
        compiler_params=pltpu.CompilerParams(use_tc_tiling_on_sc=False),
        scratch_types=[
            pltpu.VMEM((NSTREAM, 128), jnp.int32),
            pltpu.VMEM((NSTREAM, 128), jnp.int32),
            pltpu.VMEM((CHUNK, D), jnp.float32),
            pltpu.VMEM((CHUNK, D), jnp.float32),
            pltpu.VMEM((BAGS_PER_W, D), jnp.float32),
            pltpu.VMEM((OFFS_SLICE,), jnp.int32),
            pltpu.VMEM((16,), jnp.int32),
            pltpu.SemaphoreType.DMA,
            pltpu.SemaphoreType.DMA,
        ],
    )
    out_main, out2 = f(values2d, offs_pad, table)
    return out_main.at[B - 1].set(jnp.sum(out2, axis=0))


# speedup vs baseline: 198.1202x; 1.0035x over previous
import jax
import jax.numpy as jnp
from jax import lax
from jax.experimental import pallas as pl
from jax.experimental.pallas import tpu as pltpu
from jax.experimental.pallas import tpu_sc as plsc

B = 16384
D = 32
N_VALUES = B * 50
NW = 32
BAGS_PER_W = B // NW
CHUNK = 1024
NSTREAM = CHUNK // 128
OFFS_SLICE = 536
VROWS = N_VALUES // 128


def _sread(ref, i):
    return ref[pl.ds(i, 16)][0]


def _acc_rows(rows_ref, lo, hi, acc0, acc1):
    n4 = (hi - lo) // 4

    def body4(g, carry):
        a0, a1 = carry
        j = lo + g * 4
        s0 = ((rows_ref[j, pl.ds(0, 16)] + rows_ref[j + 1, pl.ds(0, 16)])
              + (rows_ref[j + 2, pl.ds(0, 16)] + rows_ref[j + 3, pl.ds(0, 16)]))
        s1 = ((rows_ref[j, pl.ds(16, 16)] + rows_ref[j + 1, pl.ds(16, 16)])
              + (rows_ref[j + 2, pl.ds(16, 16)] + rows_ref[j + 3, pl.ds(16, 16)]))
        return (a0 + s0, a1 + s1)

    acc0, acc1 = lax.fori_loop(0, n4, body4, (acc0, acc1))

    def body1(j, carry):
        a0, a1 = carry
        a0 = a0 + rows_ref[j, pl.ds(0, 16)]
        a1 = a1 + rows_ref[j, pl.ds(16, 16)]
        return (a0, a1)
    return lax.fori_loop(lo + n4 * 4, hi, body1, (acc0, acc1))


def _issue_chunk(values2d, table, idx_ref, rows_ref, sem, c0):
    r0 = pl.multiple_of(jnp.minimum(c0 // 128, VROWS - NSTREAM), NSTREAM)
    pltpu.sync_copy(values2d.at[pl.ds(r0, NSTREAM)], idx_ref)
    for j in range(NSTREAM):
        pltpu.async_copy(table.at[idx_ref.at[j]],
                         rows_ref.at[pl.ds(j * 128, 128)], sem)


def _wait_chunk(table, idx_ref, rows_ref, sem):
    for j in range(NSTREAM):
        pltpu.make_async_copy(table.at[idx_ref.at[j]],
                              rows_ref.at[pl.ds(j * 128, 128)], sem).wait()


def _sc_body(values2d, offs_hbm, table, out_hbm, out2_hbm,
             idx_a, idx_b, idx_c, rows_a, rows_b, rows_c, out_loc, offs_v,
             tail_v, sem_a, sem_b, sem_c):
    wid = lax.axis_index("s") * 2 + lax.axis_index("c")
    bag_lo = wid * BAGS_PER_W
    bag_hi = jnp.minimum(bag_lo + BAGS_PER_W, B - 1)

    pltpu.sync_copy(offs_hbm.at[pl.ds(bag_lo, OFFS_SLICE)], offs_v)
    pltpu.sync_copy(offs_hbm.at[pl.ds(B - 16, 16)], tail_v)

    zero = jnp.zeros((16,), jnp.float32)

    s_w = _sread(offs_v, 0)
    e_w = _sread(offs_v, bag_hi - bag_lo)
    a0 = (s_w // CHUNK) * CHUNK
    n_chunks = jnp.maximum((e_w - a0 + CHUNK - 1) // CHUNK, 1)
    n_loc = bag_hi - bag_lo

    def process1(c, rows_ref, carry):
        b, acc0, acc1 = carry
        c0 = a0 + c * CHUNK
        c_end = c0 + CHUNK

        def bs_body(_, st):
            lo_, hi_ = st
            mid = (lo_ + hi_) // 2
            le = _sread(offs_v, mid) <= c_end
            go = lo_ < hi_
            lo_ = jnp.where(jnp.logical_and(go, le), mid + 1, lo_)
            hi_ = jnp.where(jnp.logical_and(go, jnp.logical_not(le)),
                            mid, hi_)
            return (lo_, hi_)
        ub, _ = lax.fori_loop(0, 10, bs_body, (jnp.int32(0), jnp.int32(513)))

        b_stop = jnp.minimum(ub - 1, n_loc)

        def fin_body(ib, st):
            a0_, a1_ = st
            lo = jnp.maximum(_sread(offs_v, ib), c0) - c0
            hi = _sread(offs_v, ib + 1) - c0
            a0_, a1_ = _acc_rows(rows_ref, lo, hi, a0_, a1_)
            out_loc[ib, pl.ds(0, 16)] = a0_
            out_loc[ib, pl.ds(16, 16)] = a1_
            return (zero, zero)

        acc0, acc1 = lax.fori_loop(b, b_stop, fin_body, (acc0, acc1))
        b = jnp.maximum(b, b_stop)

        lo = jnp.maximum(_sread(offs_v, jnp.minimum(b, n_loc)), c0) - c0
        acc0, acc1 = _acc_rows(rows_ref, lo, CHUNK, acc0, acc1)
        return (b, acc0, acc1)

    bufs = ((idx_a, rows_a, sem_a), (idx_b, rows_b, sem_b),
            (idx_c, rows_c, sem_c))

    def run_pipeline(n_real, c0_of, process, carry):
        _issue_chunk(values2d, table, idx_a, rows_a, sem_a, c0_of(0))
        _issue_chunk(values2d, table, idx_b, rows_b, sem_b, c0_of(1))

        def tri_body(ci, carry):
            for k in range(3):
                c = 3 * ci + k
                i_w, r_w, s_w_ = bufs[k]
                i_n, r_n, s_n = bufs[(k + 2) % 3]
                _wait_chunk(table, i_w, r_w, s_w_)
                _issue_chunk(values2d, table, i_n, r_n, s_n, c0_of(c + 2))
                carry = process(c, r_w, carry)
            return carry

        n_tri = (n_real + 2) // 3
        carry = lax.fori_loop(0, n_tri, tri_body, carry)
        _wait_chunk(table, idx_a, rows_a, sem_a)
        _wait_chunk(table, idx_b, rows_b, sem_b)
        return carry

    run_pipeline(n_chunks, lambda c: a0 + c * CHUNK, process1,
                 (jnp.int32(0), zero, zero))
    pltpu.sync_copy(out_loc, out_hbm.at[pl.ds(bag_lo, BAGS_PER_W)])

    t0 = tail_v[...][15]
    a0t = (t0 // 128) * 128
    q = ((N_VALUES - a0t + NW * 128 - 1) // (NW * 128)) * 128
    lo_w = jnp.minimum(jnp.maximum(a0t + wid * q, t0), N_VALUES)
    hi_w = jnp.minimum(a0t + (wid + 1) * q, N_VALUES)
    base = (lo_w // CHUNK) * CHUNK
    n_ch2 = jnp.maximum((hi_w - base + CHUNK - 1) // CHUNK, 0)

    def process2(c, rows_ref, carry):
        acc0, acc1 = carry
        c0 = base + c * CHUNK
        lo = jnp.maximum(lo_w, c0) - c0
        hi = jnp.minimum(jnp.maximum(hi_w - c0, lo), CHUNK)
        return _acc_rows(rows_ref, lo, hi, acc0, acc1)

    acc0, acc1 = run_pipeline(n_ch2, lambda c: base + c * CHUNK, process2,
                              (zero, zero))
    out_loc[0, pl.ds(0, 16)] = acc0
    out_loc[0, pl.ds(16, 16)] = acc1
    pltpu.sync_copy(out_loc.at[pl.ds(0, 1)], out2_hbm.at[pl.ds(wid, 1)])


@jax.jit
def kernel(values, lengths, table):
    offsets = jnp.concatenate(
        [jnp.zeros((1,), jnp.int32), jnp.cumsum(lengths, dtype=jnp.int32)])
    offs_pad = jnp.pad(offsets, (0, (B - 8 + OFFS_SLICE) - (B + 1)),
                       mode="edge")
    values2d = values.reshape(VROWS, 128)

    mesh = plsc.VectorSubcoreMesh(core_axis_name="c", subcore_axis_name="s")
    f = pl.kernel(
        _sc_body,
        out_type=[jax.ShapeDtypeStruct((B, D), jnp.float32),
                  jax.ShapeDtypeStruct((NW, D), jnp.float32)],
        mesh=mesh,
        compiler_params=pltpu.CompilerParams(use_tc_tiling_on_sc=False),
        scratch_types=[
            pltpu.VMEM((NSTREAM, 128), jnp.int32),
            pltpu.VMEM((NSTREAM, 128), jnp.int32),
            pltpu.VMEM((NSTREAM, 128), jnp.int32),
            pltpu.VMEM((CHUNK, D), jnp.float32),
            pltpu.VMEM((CHUNK, D), jnp.float32),
            pltpu.VMEM((CHUNK, D), jnp.float32),
            pltpu.VMEM((BAGS_PER_W, D), jnp.float32),
            pltpu.VMEM((OFFS_SLICE,), jnp.int32),
            pltpu.VMEM((16,), jnp.int32),
            pltpu.SemaphoreType.DMA,
            pltpu.SemaphoreType.DMA,
            pltpu.SemaphoreType.DMA,
        ],
    )
    out_main, out2 = f(values2d, offs_pad, table)
    return out_main.at[B - 1].set(jnp.sum(out2, axis=0))
